# baseline (speedup 1.0000x reference)
import functools

import jax
import jax.numpy as jnp
from jax import lax
from jax.experimental import pallas as pl
from jax.experimental.pallas import tpu as pltpu
from jax.experimental.pallas import tpu_sc as plsc

EPS = 1e-07
NC = 2
NS = 16
LANES = 16
CH = 128


def _sc_edge_pass(n_nodes, d_feat, e_pad):
    half = d_feat // 2
    epw = e_pad // NS
    nchunk = epw // CH
    nacc = n_nodes + LANES
    rpw = (n_nodes // NS) & ~7
    tail = n_nodes - NS * rpw
    ztail = nacc - NS * rpw
    mesh = plsc.VectorSubcoreMesh(core_axis_name="c", subcore_axis_name="s")

    def _chunked(total):
        done = 0
        while done < total:
            step = min(CH, total - done)
            yield done, step
            done += step

    @functools.partial(
        pl.kernel,
        out_type=jax.ShapeDtypeStruct((NC, n_nodes, half), jnp.float32),
        mesh=mesh,
        compiler_params=pltpu.CompilerParams(use_tc_tiling_on_sc=False,
                                             needs_layout_passes=False),
        scratch_types=[
            pltpu.VMEM((nchunk, CH), jnp.int32),
            pltpu.VMEM((nchunk, CH), jnp.int32),
            pltpu.VMEM((CH, half), jnp.float32),
            pltpu.VMEM((CH, half), jnp.float32),
            pltpu.VMEM((CH, half), jnp.float32),
            pltpu.VMEM((CH, d_feat), jnp.bfloat16),
            pltpu.VMEM((CH, d_feat), jnp.bfloat16),
            pltpu.VMEM_SHARED((nacc, d_feat), jnp.bfloat16),
            pltpu.SemaphoreType.DMA,
            pltpu.SemaphoreType.DMA,
            pltpu.SemaphoreType.DMA,
        ],
    )
    def sc_kernel(xr, src, dst, out, idx_big, dst_big,
                  gbuf0, gbuf1, gbuf2, sbuf0, sbuf1, acc, dsem, gsem,
                  ssem):
        c = lax.axis_index("c")
        s = lax.axis_index("s")
        gbuf = (gbuf0, gbuf1, gbuf2)
        sbuf = (sbuf0, sbuf1)
        pk = 2 * LANES

        def zrow(r, _):
            for j in range(d_feat // pk):
                sbuf0[r, pl.ds(j * pk, pk)] = jnp.zeros(
                    (pk,), jnp.bfloat16)
            return 0
        lax.fori_loop(0, CH, zrow, 0)
        zbase = s * rpw
        for off, step in _chunked(rpw):
            pltpu.sync_copy(sbuf0.at[pl.ds(0, step)],
                            acc.at[pl.ds(zbase + off, step)])
        if ztail:
            @pl.when(s == NS - 1)
            def _():
                for off, step in _chunked(ztail):
                    pltpu.sync_copy(
                        sbuf0.at[pl.ds(0, step)],
                        acc.at[pl.ds(NS * rpw + off, step)])
        plsc.subcore_barrier()

        pltpu.sync_copy(src.at[pl.ds(s * nchunk, nchunk)], idx_big)
        pltpu.sync_copy(dst.at[pl.ds(s * nchunk, nchunk)], dst_big)

        def idx_row(r, _):
            vs = [idx_big[r, pl.ds(i * LANES, LANES)]
                  for i in range(CH // LANES)]
            ws = [v * 2 + c for v in vs]
            for i, w in enumerate(ws):
                idx_big[r, pl.ds(i * LANES, LANES)] = w
            return 0
        lax.fori_loop(0, nchunk, idx_row, 0)

        def compute_chunk2(gb, pb):
            def edge_body(e2, _):
                e0 = e2 * 8
                uj = [(u, j) for u in range(8)
                      for j in range(half // LANES)]
                vs = [gbuf[gb][e0 + u, pl.ds(j * LANES, LANES)]
                      for (u, j) in uj]
                gs = [jnp.maximum(v, 0.0) for v in vs]
                exs = [jnp.exp(g) for g in gs]
                gexs = [g * ex for g, ex in zip(gs, exs)]
                pks = [plsc.pack(ex, gex, format=plsc.PackFormat.INTERLEAVED)
                       for ex, gex in zip(exs, gexs)]
                for (u, j), pv in zip(uj, pks):
                    sbuf[pb][e0 + u, pl.ds(j * pk, pk)] = pv
                return 0
            lax.fori_loop(0, CH // 8, edge_body, 0)

        pltpu.async_copy(xr.at[idx_big.at[0]], gbuf0, gsem)
        pltpu.async_copy(xr.at[idx_big.at[1]], gbuf1, gsem)

        def sub_iter(k, g3, p2):
            @pl.when(k + 2 < nchunk)
            def _():
                pltpu.async_copy(xr.at[idx_big.at[k + 2]],
                                 gbuf[(g3 + 2) % 3], gsem)
            pltpu.make_async_copy(xr.at[idx_big.at[k]], gbuf[g3],
                                  gsem).wait()
            compute_chunk2(g3, p2)

            @pl.when(k >= 1)
            def _():
                pltpu.make_async_copy(
                    sbuf[1 - p2], acc.at[dst_big.at[k - 1]], ssem).wait()
            pltpu.async_copy(sbuf[p2], acc.at[dst_big.at[k]], ssem,
                             add=True)

        def loop_body(k6, _):
            for q in range(6):
                sub_iter(k6 * 6 + q, q % 3, q % 2)
            return 0
        lax.fori_loop(0, nchunk // 6, loop_body, 0)
        pltpu.make_async_copy(
            sbuf1, acc.at[dst_big.at[nchunk - 1]], ssem).wait()
        plsc.subcore_barrier()

        def divide_rows(t, step):
            sb, gb = sbuf[t % 2], gbuf[t % 2]

            def rbody(r2, _):
                r0 = r2 * 2
                uj = [(u, j) for u in range(2)
                      for j in range(d_feat // pk)]
                pvs = [sb[r0 + u, pl.ds(j * pk, pk)] for (u, j) in uj]
                egs = [plsc.unpack(pv, format=plsc.PackFormat.INTERLEAVED,
                                   preferred_element_type=jnp.float32)
                       for pv in pvs]
                ms = [gex / (ex + 1e-16) + EPS for ex, gex in egs]
                for (u, j), m in zip(uj, ms):
                    gb[r0 + u, pl.ds(j * LANES, LANES)] = m
                return 0
            lax.fori_loop(0, step // 2, rbody, 0)

        def emit_pipelined(jobs):
            nj = len(jobs)
            b0, s0 = jobs[0]
            pltpu.async_copy(acc.at[pl.ds(b0, s0)],
                             sbuf[0].at[pl.ds(0, s0)], dsem)
            for t, (bt, st) in enumerate(jobs):
                pltpu.make_async_copy(acc.at[pl.ds(bt, st)],
                                      sbuf[t % 2].at[pl.ds(0, st)],
                                      dsem).wait()
                if t + 1 < nj:
                    bn, sn = jobs[t + 1]
                    pltpu.async_copy(acc.at[pl.ds(bn, sn)],
                                     sbuf[(t + 1) % 2].at[pl.ds(0, sn)],
                                     dsem)
                if t >= 2:
                    bo, so = jobs[t - 2]
                    pltpu.make_async_copy(gbuf[t % 2].at[pl.ds(0, so)],
                                          out.at[c, pl.ds(bo, so)],
                                          ssem).wait()
                divide_rows(t, st)
                pltpu.async_copy(gbuf[t % 2].at[pl.ds(0, st)],
                                 out.at[c, pl.ds(bt, st)], ssem)
            for t in range(max(nj - 2, 0), nj):
                bo, so = jobs[t]
                pltpu.make_async_copy(gbuf[t % 2].at[pl.ds(0, so)],
                                      out.at[c, pl.ds(bo, so)],
                                      ssem).wait()

        obase = s * rpw
        emit_pipelined([(obase + off, step) for off, step in _chunked(rpw)])
        if tail:
            @pl.when(s == NS - 1)
            def _():
                emit_pipelined([(NS * rpw + off, step)
                                for off, step in _chunked(tail)])

    return sc_kernel


def _tc_mlp(n_nodes, d_feat, h_feat, blk):
    half = d_feat // 2
    nb = n_nodes // blk
    inv_n = 1.0 / n_nodes

    def body(x_ref, s_ref, w1_ref, b1_ref, gamma_ref, beta_ref, w2_ref,
             b2_ref, o_ref, h1_scr, sums_scr):
        p = pl.program_id(0)
        i = pl.program_id(1)

        @pl.when(p == 0)
        def _():
            m = jnp.concatenate([s_ref[0], s_ref[1]], axis=1)
            h = x_ref[...] + m
            h1 = jnp.dot(h, w1_ref[...],
                         preferred_element_type=jnp.float32) + b1_ref[...]
            h1_scr[pl.ds(i * blk, blk), :] = h1

            @pl.when(i == 0)
            def _():
                sums_scr[...] = jnp.zeros_like(sums_scr)

            upd = jnp.concatenate(
                [jnp.sum(h1, axis=0, keepdims=True),
                 jnp.sum(h1 * h1, axis=0, keepdims=True),
                 jnp.zeros((6, h_feat), jnp.float32)], axis=0)
            sums_scr[...] += upd

        @pl.when(p == 1)
        def _():
            mean = sums_scr[0:1, :] * inv_n
            var = sums_scr[1:2, :] * inv_n - mean * mean
            scale = lax.rsqrt(var + 1e-05) * gamma_ref[...]
            h1 = h1_scr[pl.ds(i * blk, blk), :]
            h1n = (h1 - mean) * scale + beta_ref[...]
            h1n = jnp.maximum(h1n, 0.0)
            o_ref[...] = jnp.dot(
                h1n, w2_ref[...],
                preferred_element_type=jnp.float32) + b2_ref[...]

    return pl.pallas_call(
        body,
        grid=(2, nb),
        in_specs=[
            pl.BlockSpec((blk, d_feat), lambda p, i: (i, 0)),
            pl.BlockSpec((NC, blk, half), lambda p, i: (0, i, 0)),
            pl.BlockSpec((d_feat, h_feat), lambda p, i: (0, 0)),
            pl.BlockSpec((1, h_feat), lambda p, i: (0, 0)),
            pl.BlockSpec((1, h_feat), lambda p, i: (0, 0)),
            pl.BlockSpec((1, h_feat), lambda p, i: (0, 0)),
            pl.BlockSpec((h_feat, d_feat), lambda p, i: (0, 0)),
            pl.BlockSpec((1, d_feat), lambda p, i: (0, 0)),
        ],
        out_specs=pl.BlockSpec((blk, d_feat), lambda p, i: (i, 0)),
        out_shape=jax.ShapeDtypeStruct((n_nodes, d_feat), jnp.float32),
        scratch_shapes=[
            pltpu.VMEM((n_nodes, h_feat), jnp.float32),
            pltpu.VMEM((8, h_feat), jnp.float32),
        ],
    )


def kernel(x, edge_index, W1, b1, gamma, beta, W2, b2):
    n, d = x.shape
    h_feat = W1.shape[1]
    e = edge_index.shape[1]

    e_pad = ((e + 6 * NS * CH - 1) // (6 * NS * CH)) * (6 * NS * CH)
    pad = e_pad - e
    src = edge_index[0]
    dst = edge_index[1]
    if pad:
        src = jnp.concatenate([src, jnp.zeros((pad,), jnp.int32)])
        dst = jnp.concatenate([dst, jnp.full((pad,), n, jnp.int32)])
    xr = x.reshape(2 * n, d // 2)

    s_acc = _sc_edge_pass(n, d, e_pad)(
        xr, src.reshape(e_pad // CH, CH), dst.reshape(e_pad // CH, CH))

    blk = 1000 if n % 1000 == 0 else n // 8
    out = _tc_mlp(n, d, h_feat, blk)(
        x, s_acc, W1, b1.reshape(1, h_feat), gamma.reshape(1, h_feat),
        beta.reshape(1, h_feat), W2, b2.reshape(1, d))
    return out


# speedup vs baseline: 1.8634x; 1.8634x over previous
import functools

import jax
import jax.numpy as jnp
from jax import lax
from jax.experimental import pallas as pl
from jax.experimental.pallas import tpu as pltpu
from jax.experimental.pallas import tpu_sc as plsc

EPS = 1e-07
NC = 2
NS = 16
LANES = 16
CH = 128


def _sc_edge_pass(n_nodes, d_feat, e_pad):
    half = d_feat // 2
    epw = e_pad // NS
    nchunk = epw // CH
    nacc = n_nodes + LANES
    rpw = (n_nodes // NS) & ~7
    tail = n_nodes - NS * rpw
    ztail = nacc - NS * rpw
    mesh = plsc.VectorSubcoreMesh(core_axis_name="c", subcore_axis_name="s")

    def _chunked(total):
        done = 0
        while done < total:
            step = min(CH, total - done)
            yield done, step
            done += step

    @functools.partial(
        pl.kernel,
        out_type=jax.ShapeDtypeStruct((NC, n_nodes, half), jnp.float32),
        mesh=mesh,
        compiler_params=pltpu.CompilerParams(use_tc_tiling_on_sc=False,
                                             needs_layout_passes=False),
        scratch_types=[
            pltpu.VMEM((nchunk, CH), jnp.int32),
            pltpu.VMEM((nchunk, CH), jnp.int32),
            pltpu.VMEM((CH, half), jnp.float32),
            pltpu.VMEM((CH, half), jnp.float32),
            pltpu.VMEM((CH, d_feat), jnp.bfloat16),
            pltpu.VMEM((CH, d_feat), jnp.bfloat16),
            pltpu.VMEM_SHARED((nacc, d_feat), jnp.bfloat16),
            pltpu.SemaphoreType.DMA,
            pltpu.SemaphoreType.DMA,
            pltpu.SemaphoreType.DMA,
        ],
    )
    def sc_kernel(xr, src, dst, out, idx_big, dst_big,
                  gbuf0, gbuf1, sbuf0, sbuf1, acc, dsem, gsem, ssem):
        c = lax.axis_index("c")
        s = lax.axis_index("s")
        gbuf = (gbuf0, gbuf1)
        sbuf = (sbuf0, sbuf1)
        pk = 2 * LANES

        def zrow(r, _):
            for j in range(d_feat // pk):
                sbuf0[r, pl.ds(j * pk, pk)] = jnp.zeros(
                    (pk,), jnp.bfloat16)
            return 0
        lax.fori_loop(0, CH, zrow, 0)
        zbase = s * rpw
        for off, step in _chunked(rpw):
            pltpu.sync_copy(sbuf0.at[pl.ds(0, step)],
                            acc.at[pl.ds(zbase + off, step)])
        if ztail:
            @pl.when(s == NS - 1)
            def _():
                for off, step in _chunked(ztail):
                    pltpu.sync_copy(
                        sbuf0.at[pl.ds(0, step)],
                        acc.at[pl.ds(NS * rpw + off, step)])
        plsc.subcore_barrier()

        pltpu.sync_copy(src.at[pl.ds(s * nchunk, nchunk)], idx_big)
        pltpu.sync_copy(dst.at[pl.ds(s * nchunk, nchunk)], dst_big)

        def idx_row(r, _):
            vs = [idx_big[r, pl.ds(i * LANES, LANES)]
                  for i in range(CH // LANES)]
            ws = [v * 2 + c for v in vs]
            for i, w in enumerate(ws):
                idx_big[r, pl.ds(i * LANES, LANES)] = w
            return 0
        lax.fori_loop(0, nchunk, idx_row, 0)

        def compute_chunk2(gb, pb):
            def edge_body(e2, _):
                e0 = e2 * 8
                uj = [(u, j) for u in range(8)
                      for j in range(half // LANES)]
                vs = [gbuf[gb][e0 + u, pl.ds(j * LANES, LANES)]
                      for (u, j) in uj]
                gs = [jnp.maximum(v, 0.0) for v in vs]
                exs = [jnp.exp(g) for g in gs]
                gexs = [g * ex for g, ex in zip(gs, exs)]
                pks = [plsc.pack(ex, gex, format=plsc.PackFormat.INTERLEAVED)
                       for ex, gex in zip(exs, gexs)]
                for (u, j), pv in zip(uj, pks):
                    sbuf[pb][e0 + u, pl.ds(j * pk, pk)] = pv
                return 0
            lax.fori_loop(0, CH // 8, edge_body, 0)

        pltpu.async_copy(xr.at[idx_big.at[0]], gbuf0, gsem)
        pltpu.async_copy(xr.at[idx_big.at[1]], gbuf1, gsem)

        def sub_iter(k, a, b):
            pltpu.make_async_copy(xr.at[idx_big.at[k]], gbuf[a],
                                  gsem).wait()
            compute_chunk2(a, a)

            @pl.when(k >= 1)
            def _():
                pltpu.make_async_copy(
                    sbuf[b], acc.at[dst_big.at[k - 1]], ssem).wait()
            pltpu.async_copy(sbuf[a], acc.at[dst_big.at[k]], ssem,
                             add=True)

            @pl.when(k + 2 < nchunk)
            def _():
                pltpu.async_copy(xr.at[idx_big.at[k + 2]], gbuf[a], gsem)

        def loop_body(k2, _):
            sub_iter(k2 * 2, 0, 1)
            sub_iter(k2 * 2 + 1, 1, 0)
            return 0
        lax.fori_loop(0, nchunk // 2, loop_body, 0)
        pltpu.make_async_copy(
            sbuf1, acc.at[dst_big.at[nchunk - 1]], ssem).wait()
        plsc.subcore_barrier()

        def divide_rows(t, step):
            sb, gb = sbuf[t % 2], gbuf[t % 2]

            def rbody(r2, _):
                r0 = r2 * 2
                uj = [(u, j) for u in range(2)
                      for j in range(d_feat // pk)]
                pvs = [sb[r0 + u, pl.ds(j * pk, pk)] for (u, j) in uj]
                egs = [plsc.unpack(pv, format=plsc.PackFormat.INTERLEAVED,
                                   preferred_element_type=jnp.float32)
                       for pv in pvs]
                ms = [gex / (ex + 1e-16) + EPS for ex, gex in egs]
                for (u, j), m in zip(uj, ms):
                    gb[r0 + u, pl.ds(j * LANES, LANES)] = m
                return 0
            lax.fori_loop(0, step // 2, rbody, 0)

        def emit_pipelined(jobs):
            nj = len(jobs)
            b0, s0 = jobs[0]
            pltpu.async_copy(acc.at[pl.ds(b0, s0)],
                             sbuf[0].at[pl.ds(0, s0)], dsem)
            for t, (bt, st) in enumerate(jobs):
                pltpu.make_async_copy(acc.at[pl.ds(bt, st)],
                                      sbuf[t % 2].at[pl.ds(0, st)],
                                      dsem).wait()
                if t + 1 < nj:
                    bn, sn = jobs[t + 1]
                    pltpu.async_copy(acc.at[pl.ds(bn, sn)],
                                     sbuf[(t + 1) % 2].at[pl.ds(0, sn)],
                                     dsem)
                if t >= 2:
                    bo, so = jobs[t - 2]
                    pltpu.make_async_copy(gbuf[t % 2].at[pl.ds(0, so)],
                                          out.at[c, pl.ds(bo, so)],
                                          ssem).wait()
                divide_rows(t, st)
                pltpu.async_copy(gbuf[t % 2].at[pl.ds(0, st)],
                                 out.at[c, pl.ds(bt, st)], ssem)
            for t in range(max(nj - 2, 0), nj):
                bo, so = jobs[t]
                pltpu.make_async_copy(gbuf[t % 2].at[pl.ds(0, so)],
                                      out.at[c, pl.ds(bo, so)],
                                      ssem).wait()

        obase = s * rpw
        emit_pipelined([(obase + off, step) for off, step in _chunked(rpw)])
        if tail:
            @pl.when(s == NS - 1)
            def _():
                emit_pipelined([(NS * rpw + off, step)
                                for off, step in _chunked(tail)])

    return sc_kernel


def _tc_mlp(n_nodes, d_feat, h_feat, blk):
    half = d_feat // 2
    nb = n_nodes // blk
    inv_n = 1.0 / n_nodes

    def body(x_ref, s_ref, w1_ref, b1_ref, gamma_ref, beta_ref, w2_ref,
             b2_ref, o_ref, h1_scr, sums_scr):
        p = pl.program_id(0)
        i = pl.program_id(1)

        @pl.when(p == 0)
        def _():
            m = jnp.concatenate([s_ref[0], s_ref[1]], axis=1)
            h = x_ref[...] + m
            h1 = jnp.dot(h, w1_ref[...],
                         preferred_element_type=jnp.float32) + b1_ref[...]
            h1_scr[pl.ds(i * blk, blk), :] = h1

            @pl.when(i == 0)
            def _():
                sums_scr[...] = jnp.zeros_like(sums_scr)

            upd = jnp.concatenate(
                [jnp.sum(h1, axis=0, keepdims=True),
                 jnp.sum(h1 * h1, axis=0, keepdims=True),
                 jnp.zeros((6, h_feat), jnp.float32)], axis=0)
            sums_scr[...] += upd

        @pl.when(p == 1)
        def _():
            mean = sums_scr[0:1, :] * inv_n
            var = sums_scr[1:2, :] * inv_n - mean * mean
            scale = lax.rsqrt(var + 1e-05) * gamma_ref[...]
            h1 = h1_scr[pl.ds(i * blk, blk), :]
            h1n = (h1 - mean) * scale + beta_ref[...]
            h1n = jnp.maximum(h1n, 0.0)
            o_ref[...] = jnp.dot(
                h1n, w2_ref[...],
                preferred_element_type=jnp.float32) + b2_ref[...]

    return pl.pallas_call(
        body,
        grid=(2, nb),
        in_specs=[
            pl.BlockSpec((blk, d_feat), lambda p, i: (i, 0)),
            pl.BlockSpec((NC, blk, half), lambda p, i: (0, i, 0)),
            pl.BlockSpec((d_feat, h_feat), lambda p, i: (0, 0)),
            pl.BlockSpec((1, h_feat), lambda p, i: (0, 0)),
            pl.BlockSpec((1, h_feat), lambda p, i: (0, 0)),
            pl.BlockSpec((1, h_feat), lambda p, i: (0, 0)),
            pl.BlockSpec((h_feat, d_feat), lambda p, i: (0, 0)),
            pl.BlockSpec((1, d_feat), lambda p, i: (0, 0)),
        ],
        out_specs=pl.BlockSpec((blk, d_feat), lambda p, i: (i, 0)),
        out_shape=jax.ShapeDtypeStruct((n_nodes, d_feat), jnp.float32),
        scratch_shapes=[
            pltpu.VMEM((n_nodes, h_feat), jnp.float32),
            pltpu.VMEM((8, h_feat), jnp.float32),
        ],
    )


def kernel(x, edge_index, W1, b1, gamma, beta, W2, b2):
    n, d = x.shape
    h_feat = W1.shape[1]
    e = edge_index.shape[1]

    e_pad = ((e + 2 * NS * CH - 1) // (2 * NS * CH)) * (2 * NS * CH)
    pad = e_pad - e
    src = edge_index[0]
    dst = edge_index[1]
    if pad:
        src = jnp.concatenate([src, jnp.zeros((pad,), jnp.int32)])
        dst = jnp.concatenate([dst, jnp.full((pad,), n, jnp.int32)])
    xr = x.reshape(2 * n, d // 2)

    s_acc = _sc_edge_pass(n, d, e_pad)(
        xr, src.reshape(e_pad // CH, CH), dst.reshape(e_pad // CH, CH))

    blk = 1000 if n % 1000 == 0 else n // 8
    out = _tc_mlp(n, d, h_feat, blk)(
        x, s_acc, W1, b1.reshape(1, h_feat), gamma.reshape(1, h_feat),
        beta.reshape(1, h_feat), W2, b2.reshape(1, d))
    return out


# speedup vs baseline: 1.8663x; 1.0016x over previous
import functools

import jax
import jax.numpy as jnp
from jax import lax
from jax.experimental import pallas as pl
from jax.experimental.pallas import tpu as pltpu
from jax.experimental.pallas import tpu_sc as plsc

EPS = 1e-07
NC = 2
NS = 16
LANES = 16
CH = 128


def _sc_edge_pass(n_nodes, d_feat, e_pad):
    half = d_feat // 2
    epw = e_pad // NS
    nchunk = epw // CH
    nacc = n_nodes + LANES
    rpw = (n_nodes // NS) & ~7
    tail = n_nodes - NS * rpw
    ztail = nacc - NS * rpw
    mesh = plsc.VectorSubcoreMesh(core_axis_name="c", subcore_axis_name="s")

    def _chunked(total):
        done = 0
        while done < total:
            step = min(CH, total - done)
            yield done, step
            done += step

    @functools.partial(
        pl.kernel,
        out_type=jax.ShapeDtypeStruct((NC, n_nodes, half), jnp.float32),
        mesh=mesh,
        compiler_params=pltpu.CompilerParams(use_tc_tiling_on_sc=False,
                                             needs_layout_passes=False),
        scratch_types=[
            pltpu.VMEM((nchunk, CH), jnp.int32),
            pltpu.VMEM((nchunk, CH), jnp.int32),
            pltpu.VMEM((CH, half), jnp.float32),
            pltpu.VMEM((CH, half), jnp.float32),
            pltpu.VMEM((CH, d_feat), jnp.bfloat16),
            pltpu.VMEM((CH, d_feat), jnp.bfloat16),
            pltpu.VMEM_SHARED((nacc, d_feat), jnp.bfloat16),
            pltpu.SemaphoreType.DMA,
            pltpu.SemaphoreType.DMA,
            pltpu.SemaphoreType.DMA,
        ],
    )
    def sc_kernel(xr, src, dst, out, idx_big, dst_big,
                  gbuf0, gbuf1, sbuf0, sbuf1, acc, dsem, gsem, ssem):
        c = lax.axis_index("c")
        s = lax.axis_index("s")
        gbuf = (gbuf0, gbuf1)
        sbuf = (sbuf0, sbuf1)
        pk = 2 * LANES

        def zrow(r, _):
            for j in range(d_feat // pk):
                sbuf0[r, pl.ds(j * pk, pk)] = jnp.zeros(
                    (pk,), jnp.bfloat16)
            return 0
        lax.fori_loop(0, CH, zrow, 0)
        zbase = s * rpw
        for off, step in _chunked(rpw):
            pltpu.sync_copy(sbuf0.at[pl.ds(0, step)],
                            acc.at[pl.ds(zbase + off, step)])
        if ztail:
            @pl.when(s == NS - 1)
            def _():
                for off, step in _chunked(ztail):
                    pltpu.sync_copy(
                        sbuf0.at[pl.ds(0, step)],
                        acc.at[pl.ds(NS * rpw + off, step)])
        plsc.subcore_barrier()

        pltpu.sync_copy(src.at[pl.ds(s * nchunk, nchunk)], idx_big)
        pltpu.sync_copy(dst.at[pl.ds(s * nchunk, nchunk)], dst_big)

        def idx_row(r, _):
            vs = [idx_big[r, pl.ds(i * LANES, LANES)]
                  for i in range(CH // LANES)]
            ws = [v * 2 + c for v in vs]
            for i, w in enumerate(ws):
                idx_big[r, pl.ds(i * LANES, LANES)] = w
            return 0
        lax.fori_loop(0, nchunk, idx_row, 0)

        def compute_chunk2(gb, pb):
            def edge_body(e2, _):
                e0 = e2 * 8
                uj = [(u, j) for u in range(8)
                      for j in range(half // LANES)]
                vs = [gbuf[gb][e0 + u, pl.ds(j * LANES, LANES)]
                      for (u, j) in uj]
                gs = [jnp.maximum(v, 0.0) for v in vs]
                exs = [jnp.exp(g) for g in gs]
                gexs = [g * ex for g, ex in zip(gs, exs)]
                pks = [plsc.pack(ex, gex, format=plsc.PackFormat.INTERLEAVED)
                       for ex, gex in zip(exs, gexs)]
                for (u, j), pv in zip(uj, pks):
                    sbuf[pb][e0 + u, pl.ds(j * pk, pk)] = pv
                return 0
            lax.fori_loop(0, CH // 8, edge_body, 0)

        pltpu.async_copy(xr.at[idx_big.at[0]], gbuf0, gsem)
        pltpu.async_copy(xr.at[idx_big.at[1]], gbuf1, gsem)

        def sub_iter(k, a, b):
            pltpu.make_async_copy(xr.at[idx_big.at[k]], gbuf[a],
                                  gsem).wait()
            compute_chunk2(a, a)

            @pl.when(k + 2 < nchunk)
            def _():
                pltpu.async_copy(xr.at[idx_big.at[k + 2]], gbuf[a], gsem)

            @pl.when(k >= 1)
            def _():
                pltpu.make_async_copy(
                    sbuf[b], acc.at[dst_big.at[k - 1]], ssem).wait()
            pltpu.async_copy(sbuf[a], acc.at[dst_big.at[k]], ssem,
                             add=True)

        def loop_body(k2, _):
            sub_iter(k2 * 2, 0, 1)
            sub_iter(k2 * 2 + 1, 1, 0)
            return 0
        lax.fori_loop(0, nchunk // 2, loop_body, 0)
        pltpu.make_async_copy(
            sbuf1, acc.at[dst_big.at[nchunk - 1]], ssem).wait()
        plsc.subcore_barrier()

        def divide_rows(t, step):
            sb, gb = sbuf[t % 2], gbuf[t % 2]

            def rbody(r2, _):
                r0 = r2 * 2
                uj = [(u, j) for u in range(2)
                      for j in range(d_feat // pk)]
                pvs = [sb[r0 + u, pl.ds(j * pk, pk)] for (u, j) in uj]
                egs = [plsc.unpack(pv, format=plsc.PackFormat.INTERLEAVED,
                                   preferred_element_type=jnp.float32)
                       for pv in pvs]
                ms = [gex / (ex + 1e-16) + EPS for ex, gex in egs]
                for (u, j), m in zip(uj, ms):
                    gb[r0 + u, pl.ds(j * LANES, LANES)] = m
                return 0
            lax.fori_loop(0, step // 2, rbody, 0)

        def emit_pipelined(jobs):
            nj = len(jobs)
            b0, s0 = jobs[0]
            pltpu.async_copy(acc.at[pl.ds(b0, s0)],
                             sbuf[0].at[pl.ds(0, s0)], dsem)
            for t, (bt, st) in enumerate(jobs):
                pltpu.make_async_copy(acc.at[pl.ds(bt, st)],
                                      sbuf[t % 2].at[pl.ds(0, st)],
                                      dsem).wait()
                if t + 1 < nj:
                    bn, sn = jobs[t + 1]
                    pltpu.async_copy(acc.at[pl.ds(bn, sn)],
                                     sbuf[(t + 1) % 2].at[pl.ds(0, sn)],
                                     dsem)
                if t >= 2:
                    bo, so = jobs[t - 2]
                    pltpu.make_async_copy(gbuf[t % 2].at[pl.ds(0, so)],
                                          out.at[c, pl.ds(bo, so)],
                                          ssem).wait()
                divide_rows(t, st)
                pltpu.async_copy(gbuf[t % 2].at[pl.ds(0, st)],
                                 out.at[c, pl.ds(bt, st)], ssem)
            for t in range(max(nj - 2, 0), nj):
                bo, so = jobs[t]
                pltpu.make_async_copy(gbuf[t % 2].at[pl.ds(0, so)],
                                      out.at[c, pl.ds(bo, so)],
                                      ssem).wait()

        obase = s * rpw
        emit_pipelined([(obase + off, step) for off, step in _chunked(rpw)])
        if tail:
            @pl.when(s == NS - 1)
            def _():
                emit_pipelined([(NS * rpw + off, step)
                                for off, step in _chunked(tail)])

    return sc_kernel


def _tc_mlp(n_nodes, d_feat, h_feat, blk):
    half = d_feat // 2
    nb = n_nodes // blk
    inv_n = 1.0 / n_nodes

    def body(x_ref, s_ref, w1_ref, b1_ref, gamma_ref, beta_ref, w2_ref,
             b2_ref, o_ref, h1_scr, sums_scr):
        p = pl.program_id(0)
        i = pl.program_id(1)

        @pl.when(p == 0)
        def _():
            m = jnp.concatenate([s_ref[0], s_ref[1]], axis=1)
            h = x_ref[...] + m
            h1 = jnp.dot(h, w1_ref[...],
                         preferred_element_type=jnp.float32) + b1_ref[...]
            h1_scr[pl.ds(i * blk, blk), :] = h1

            @pl.when(i == 0)
            def _():
                sums_scr[...] = jnp.zeros_like(sums_scr)

            upd = jnp.concatenate(
                [jnp.sum(h1, axis=0, keepdims=True),
                 jnp.sum(h1 * h1, axis=0, keepdims=True),
                 jnp.zeros((6, h_feat), jnp.float32)], axis=0)
            sums_scr[...] += upd

        @pl.when(p == 1)
        def _():
            mean = sums_scr[0:1, :] * inv_n
            var = sums_scr[1:2, :] * inv_n - mean * mean
            scale = lax.rsqrt(var + 1e-05) * gamma_ref[...]
            h1 = h1_scr[pl.ds(i * blk, blk), :]
            h1n = (h1 - mean) * scale + beta_ref[...]
            h1n = jnp.maximum(h1n, 0.0)
            o_ref[...] = jnp.dot(
                h1n, w2_ref[...],
                preferred_element_type=jnp.float32) + b2_ref[...]

    return pl.pallas_call(
        body,
        grid=(2, nb),
        in_specs=[
            pl.BlockSpec((blk, d_feat), lambda p, i: (i, 0)),
            pl.BlockSpec((NC, blk, half), lambda p, i: (0, i, 0)),
            pl.BlockSpec((d_feat, h_feat), lambda p, i: (0, 0)),
            pl.BlockSpec((1, h_feat), lambda p, i: (0, 0)),
            pl.BlockSpec((1, h_feat), lambda p, i: (0, 0)),
            pl.BlockSpec((1, h_feat), lambda p, i: (0, 0)),
            pl.BlockSpec((h_feat, d_feat), lambda p, i: (0, 0)),
            pl.BlockSpec((1, d_feat), lambda p, i: (0, 0)),
        ],
        out_specs=pl.BlockSpec((blk, d_feat), lambda p, i: (i, 0)),
        out_shape=jax.ShapeDtypeStruct((n_nodes, d_feat), jnp.float32),
        scratch_shapes=[
            pltpu.VMEM((n_nodes, h_feat), jnp.float32),
            pltpu.VMEM((8, h_feat), jnp.float32),
        ],
    )


def kernel(x, edge_index, W1, b1, gamma, beta, W2, b2):
    n, d = x.shape
    h_feat = W1.shape[1]
    e = edge_index.shape[1]

    e_pad = ((e + 2 * NS * CH - 1) // (2 * NS * CH)) * (2 * NS * CH)
    pad = e_pad - e
    src = edge_index[0]
    dst = edge_index[1]
    if pad:
        src = jnp.concatenate([src, jnp.zeros((pad,), jnp.int32)])
        dst = jnp.concatenate([dst, jnp.full((pad,), n, jnp.int32)])
    xr = x.reshape(2 * n, d // 2)

    s_acc = _sc_edge_pass(n, d, e_pad)(
        xr, src.reshape(e_pad // CH, CH), dst.reshape(e_pad // CH, CH))

    blk = 1000 if n % 1000 == 0 else n // 8
    out = _tc_mlp(n, d, h_feat, blk)(
        x, s_acc, W1, b1.reshape(1, h_feat), gamma.reshape(1, h_feat),
        beta.reshape(1, h_feat), W2, b2.reshape(1, d))
    return out
